# baseline (speedup 1.0000x reference)
import functools

import jax
import jax.numpy as jnp
from jax import lax
from jax.experimental import pallas as pl
from jax.experimental.pallas import tpu as pltpu
from jax.experimental.pallas import tpu_sc as plsc

_N = 10000
_E = 320000
_D = 128
_HD = _D // 2
_NB = 15
_EPS = 1e-5

_NC = 2
_NS = 16
_CH = 128
_NCH = 160
_EPT = _NCH * _CH
_EPAD = _EPT * _NS
_NP = 10240
_RPT = _NP // _NS


def _spmv_body(y_hbm, colrow_hbm, val_hbm, out_hbm,
               m0, m1, m2, m3, m4, m5, m6, m7,
               v0, v1, v2, v3, v4, v5, v6, v7,
               g0, g1, g2, g3, s0, s1, s2, s3,
               i0, i1, i2, i3, r0, r1, r2, r3, acc,
               semm0, semm1, semm2, semm3, semm4, semm5, semm6, semm7,
               semg0, semg1, semg2, semg3, sems0, sems1, sems2, sems3):
    c = lax.axis_index("c")
    s = lax.axis_index("s")
    mb = (m0, m1, m2, m3, m4, m5, m6, m7)
    vb = (v0, v1, v2, v3, v4, v5, v6, v7)
    gbuf = (g0, g1, g2, g3)
    sbuf = (s0, s1, s2, s3)
    ib = (i0, i1, i2, i3)
    rb = (r0, r1, r2, r3)
    semm = (semm0, semm1, semm2, semm3, semm4, semm5, semm6, semm7)
    semg = (semg0, semg1, semg2, semg3)
    sems = (sems0, sems1, sems2, sems3)
    coff = c * _N

    def meta_start(ch, m):
        pltpu.make_async_copy(colrow_hbm.at[s, ch], mb[m], semm[m]).start()
        pltpu.make_async_copy(val_hbm.at[s, ch], vb[m], semm[m]).start()

    def meta_wait(ch, m):
        pltpu.make_async_copy(colrow_hbm.at[s, ch], mb[m], semm[m]).wait()
        pltpu.make_async_copy(val_hbm.at[s, ch], vb[m], semm[m]).wait()

    def gather_desc(h):
        return pltpu.make_async_copy(y_hbm.at[ib[h]], gbuf[h], semg[h])

    def scatter_desc(h):
        return pltpu.make_async_copy(sbuf[h], acc.at[rb[h]], sems[h])

    def build_idx(h, m):
        for g in range(_CH // 16):
            sl = pl.ds(g * 16, 16)
            ib[h][sl] = mb[m][0, sl] + coff

    def build_rows(h, m):
        for g in range(_CH // 16):
            sl = pl.ds(g * 16, 16)
            rb[h][sl] = mb[m][1, sl]

    def scale(h, m):
        def body(g, carry):
            v16 = vb[m][pl.ds(g * 16, 16)]
            for e in range(16):
                bidx = jnp.full((16,), e, jnp.int32)
                v = jnp.take_along_axis(v16, bidx, axis=0,
                                        mode="promise_in_bounds")
                for j in range(_HD // 16):
                    sl = pl.ds(j * 16, 16)
                    sbuf[h][g * 16 + e, sl] = gbuf[h][g * 16 + e, sl] * v
            return carry

        lax.fori_loop(0, 8, body, 0)

    for j in range(6):
        meta_start(j, j)
    for u in range(4):
        meta_wait(u, u)
        build_idx(u, u)
        gather_desc(u).start()

    def zrow(r, carry):
        for j in range(_HD // 16):
            s0[r, pl.ds(j * 16, 16)] = jnp.zeros((16,), jnp.float32)
        return carry

    lax.fori_loop(0, _CH, zrow, 0)
    base = pl.multiple_of(s * _RPT, 8)
    for k in range(_RPT // _CH):
        pltpu.sync_copy(s0, acc.at[pl.ds(base + k * _CH, _CH)])
    plsc.subcore_barrier()

    def step(k, carry):
        for u in range(8):
            ch = 8 * k + u
            h = u % 4
            gather_desc(h).wait()

            @pl.when(ch >= 4)
            def _():
                scatter_desc(h).wait()

            @pl.when(ch + 6 < _NCH)
            def _():
                meta_start(ch + 6, (u + 6) % 8)

            scale(h, u)
            build_rows(h, u)

            @pl.when(ch + 4 < _NCH)
            def _():
                meta_wait(ch + 4, (u + 4) % 8)
                build_idx(h, (u + 4) % 8)
                gather_desc(h).start()

            scatter_desc(h).start(add=True)
        return carry

    lax.fori_loop(0, _NCH // 8, step, 0)
    for u in range(4):
        scatter_desc(u).wait()
    plsc.subcore_barrier()
    pltpu.sync_copy(acc.at[pl.ds(base, _RPT)],
                    out_hbm.at[c, pl.ds(base, _RPT)])


@functools.cache
def _make_spmv_call():
    return pl.kernel(
        _spmv_body,
        out_type=jax.ShapeDtypeStruct((_NC, _NP, _HD), jnp.float32),
        mesh=plsc.VectorSubcoreMesh(core_axis_name="c", subcore_axis_name="s",
                                    num_cores=_NC, num_subcores=_NS),
        scratch_types=(
            [pltpu.VMEM((2, _CH), jnp.int32)] * 8
            + [pltpu.VMEM((_CH,), jnp.float32)] * 8
            + [pltpu.VMEM((_CH, _HD), jnp.float32)] * 8
            + [pltpu.VMEM((_CH,), jnp.int32)] * 8
            + [pltpu.VMEM_SHARED((_NP, _HD), jnp.float32)]
            + [pltpu.SemaphoreType.DMA] * 16
        ),
        compiler_params=pltpu.CompilerParams(use_tc_tiling_on_sc=False),
        name="spmv_sc",
    )


def _spmv_call(*args):
    return _make_spmv_call()(*args)


def _elu(v):
    return jnp.where(v > 0, v, jnp.exp(jnp.minimum(v, 0.0)) - 1.0)


def _stats(y):
    m = jnp.sum(y, axis=0, keepdims=True) * (1.0 / _N)
    v = jnp.sum(y * y, axis=0, keepdims=True) * (1.0 / _N) - m * m
    return m, v


def _fold(m, v, g, b):
    s = g * lax.rsqrt(v + _EPS)
    return s, b - m * s


def _emit_y(y_ref, e):
    y_ref[0:_N, :] = e[:, :_HD]
    y_ref[_N:2 * _N, :] = e[:, _HD:]


def _read_y(y_ref):
    return jnp.concatenate([y_ref[0:_N], y_ref[_N:2 * _N]], axis=1)


def _read_other(part_ref):
    return jnp.concatenate([part_ref[0, :_N], part_ref[1, :_N]], axis=1)


def _seg0_body(inp_ref, w_ref, b_ref, x0_ref, y_ref):
    x = jnp.dot(inp_ref[...], w_ref[...],
                preferred_element_type=jnp.float32) + b_ref[...]
    x0_ref[...] = x
    _emit_y(y_ref, _elu(x))


def _sub_even(y, other, wt_ref, wb_ref, fcb_ref, g1_ref, b1_ref, g2_ref, b2_ref):
    m1, v1 = _stats(y)
    m2, v2 = _stats(other)
    s1, sh1 = _fold(m1, v1, g1_ref[...], b1_ref[...])
    s2, sh2 = _fold(m2, v2, g2_ref[...], b2_ref[...])
    sh = (jnp.dot(sh1, wt_ref[...], preferred_element_type=jnp.float32)
          + jnp.dot(sh2, wb_ref[...], preferred_element_type=jnp.float32)
          + fcb_ref[...])
    return (jnp.dot(y * s1, wt_ref[...], preferred_element_type=jnp.float32)
            + jnp.dot(other * s2, wb_ref[...], preferred_element_type=jnp.float32)
            + sh)


def _sub_odd(x, wt_ref, wb_ref, fcb_ref, g1_ref, b1_ref, b2_ref):
    y = _elu(x)
    m1, v1 = _stats(y)
    s1, sh1 = _fold(m1, v1, g1_ref[...], b1_ref[...])
    sh = (jnp.dot(sh1, wt_ref[...], preferred_element_type=jnp.float32)
          + jnp.dot(b2_ref[...], wb_ref[...], preferred_element_type=jnp.float32)
          + fcb_ref[...])
    return jnp.dot(y * s1, wt_ref[...], preferred_element_type=jnp.float32) + sh


def _seg_mid_body(y_ref, part_ref, wt_ref, wb_ref, fcb_ref,
                  g1_ref, b1_ref, g2_ref, b2_ref, y2_ref):
    x = _sub_even(_read_y(y_ref), _read_other(part_ref), wt_ref, wb_ref,
                  fcb_ref, g1_ref, b1_ref, g2_ref, b2_ref)
    _emit_y(y2_ref, _elu(x))


def _seg_fin_body(y2_ref, part_ref, res_ref, wt_ref, wb_ref, fcb_ref,
                  g1_ref, b1_ref, g2_ref, b2_ref,
                  owt0_ref, owb0_ref, ofcb0_ref, og0_ref, obt0_ref, obb0_ref,
                  owt1_ref, owb1_ref, ofcb1_ref, og1_ref, obt1_ref, obb1_ref,
                  xn_ref, yn_ref):
    x = _sub_even(_read_y(y2_ref), _read_other(part_ref), wt_ref, wb_ref,
                  fcb_ref, g1_ref, b1_ref, g2_ref, b2_ref)
    x = x + res_ref[...]
    r2 = x
    x = _sub_odd(x, owt0_ref, owb0_ref, ofcb0_ref, og0_ref, obt0_ref, obb0_ref)
    x = _sub_odd(x, owt1_ref, owb1_ref, ofcb1_ref, og1_ref, obt1_ref, obb1_ref)
    x = x + r2
    xn_ref[...] = x
    _emit_y(yn_ref, _elu(x))


def _seg_last_body(y2_ref, part_ref, res_ref, wt_ref, wb_ref, fcb_ref,
                   g1_ref, b1_ref, g2_ref, b2_ref,
                   c2g_ref, c2b_ref, c2w_ref, c2bias_ref, out_ref):
    x = _sub_even(_read_y(y2_ref), _read_other(part_ref), wt_ref, wb_ref,
                  fcb_ref, g1_ref, b1_ref, g2_ref, b2_ref)
    x = x + res_ref[...]
    m, v = _stats(x)
    s, sh = _fold(m, v, c2g_ref[...], c2b_ref[...])
    out = (jnp.dot(x * s, c2w_ref[...], preferred_element_type=jnp.float32)
           + jnp.dot(sh, c2w_ref[...], preferred_element_type=jnp.float32)
           + c2bias_ref[...])
    out_ref[...] = _elu(out)


_f32 = jnp.float32
_nd = jax.ShapeDtypeStruct((_N, _D), _f32)
_ys = jax.ShapeDtypeStruct((2 * _N, _HD), _f32)

_seg0_call = pl.pallas_call(_seg0_body, out_shape=[_nd, _ys], name="seg0")
_seg_mid_call = pl.pallas_call(_seg_mid_body, out_shape=[_ys], name="seg_mid")
_seg_fin_call = pl.pallas_call(_seg_fin_body, out_shape=[_nd, _ys], name="seg_fin")
_seg_last_call = pl.pallas_call(_seg_last_body, out_shape=[_nd], name="seg_last")


def kernel(L, L_values, mask, inputs, conv1_W, conv1_b, rn_bn_gamma, rn_bn_beta,
           rn_fc_W, rn_fc_b, conv2_bn_gamma, conv2_bn_beta, conv2_W, conv2_b):
    f32 = jnp.float32
    row = L[0].astype(jnp.int32)
    col = L[1].astype(jnp.int32)
    vals = L_values.astype(f32)
    col, row, vals = lax.sort([col, row, vals], num_keys=1)
    pad = _EPAD - _E
    colp = jnp.concatenate([col, jnp.zeros((pad,), jnp.int32)]).reshape(_NS, _NCH, _CH)
    rowp = jnp.concatenate([row, jnp.zeros((pad,), jnp.int32)]).reshape(_NS, _NCH, _CH)
    valp = jnp.concatenate([vals, jnp.zeros((pad,), f32)]).reshape(_NS, _NCH, _CH)
    colrow = jnp.stack([colp, rowp], axis=2)

    inp = jnp.pad(inputs[0].astype(f32), ((0, 0), (0, 2)))
    w1 = jnp.pad(conv1_W.astype(f32), ((0, 2), (0, 0)))

    def even_args(i, j):
        return (rn_fc_W[i, j, :_D, :], rn_fc_W[i, j, _D:, :],
                rn_fc_b[i, j].reshape(1, _D),
                rn_bn_gamma[i, j, :_D].reshape(1, _D),
                rn_bn_beta[i, j, :_D].reshape(1, _D),
                rn_bn_gamma[i, j, _D:].reshape(1, _D),
                rn_bn_beta[i, j, _D:].reshape(1, _D))

    def odd_args(i, j):
        return (rn_fc_W[i, j, :_D, :], rn_fc_W[i, j, _D:, :],
                rn_fc_b[i, j].reshape(1, _D),
                rn_bn_gamma[i, j, :_D].reshape(1, _D),
                rn_bn_beta[i, j, :_D].reshape(1, _D),
                rn_bn_beta[i, j, _D:].reshape(1, _D))

    x0, y = _seg0_call(inp, w1, conv1_b.reshape(1, _D))
    res = x0
    out = None
    for i in range(0, _NB, 2):
        part = _spmv_call(y, colrow, valp)
        (y2,) = _seg_mid_call(y, part, *even_args(i, 0))
        part2 = _spmv_call(y2, colrow, valp)
        if i + 1 < _NB:
            res, y = _seg_fin_call(y2, part2, res, *even_args(i, 1),
                                   *odd_args(i + 1, 0), *odd_args(i + 1, 1))
        else:
            c2w = jnp.pad(conv2_W.astype(f32), ((0, 0), (0, _D - conv2_W.shape[1])))
            c2bias = jnp.pad(conv2_b.astype(f32), (0, _D - conv2_b.shape[0]))
            (out,) = _seg_last_call(y2, part2, res, *even_args(i, 1),
                                    conv2_bn_gamma.reshape(1, _D),
                                    conv2_bn_beta.reshape(1, _D),
                                    c2w, c2bias.reshape(1, _D))
    return out[:, :1].reshape(1, _N, 1)


# speedup vs baseline: 1.2029x; 1.2029x over previous
import functools

import jax
import jax.numpy as jnp
from jax import lax
from jax.experimental import pallas as pl
from jax.experimental.pallas import tpu as pltpu
from jax.experimental.pallas import tpu_sc as plsc

_N = 10000
_E = 320000
_D = 128
_HD = _D // 2
_NB = 15
_EPS = 1e-5

_NC = 2
_NS = 16
_CH = 128
_NCH = 160
_EPT = _NCH * _CH
_EPAD = _EPT * _NS
_NP = 10240
_RPT = _NP // _NS


def _spmv_body(y_hbm, colrow_hbm, val_hbm, out_hbm,
               m0, m1, m2, m3, m4, m5, m6, m7,
               v0, v1, v2, v3, v4, v5, v6, v7,
               g0, g1, g2, g3, s0, s1, s2, s3,
               i0, i1, i2, i3, r0, r1, r2, r3, acc,
               semm0, semm1, semm2, semm3, semm4, semm5, semm6, semm7,
               semg0, semg1, semg2, semg3, sems0, sems1, sems2, sems3):
    c = lax.axis_index("c")
    s = lax.axis_index("s")
    mb = (m0, m1, m2, m3, m4, m5, m6, m7)
    vb = (v0, v1, v2, v3, v4, v5, v6, v7)
    gbuf = (g0, g1, g2, g3)
    sbuf = (s0, s1, s2, s3)
    ib = (i0, i1, i2, i3)
    rb = (r0, r1, r2, r3)
    semm = (semm0, semm1, semm2, semm3, semm4, semm5, semm6, semm7)
    semg = (semg0, semg1, semg2, semg3)
    sems = (sems0, sems1, sems2, sems3)
    coff = c * _N

    def meta_start(ch, m):
        pltpu.make_async_copy(colrow_hbm.at[s, ch], mb[m], semm[m]).start()
        pltpu.make_async_copy(val_hbm.at[s, ch], vb[m], semm[m]).start()

    def meta_wait(ch, m):
        pltpu.make_async_copy(colrow_hbm.at[s, ch], mb[m], semm[m]).wait()
        pltpu.make_async_copy(val_hbm.at[s, ch], vb[m], semm[m]).wait()

    def gather_desc(h):
        return pltpu.make_async_copy(y_hbm.at[ib[h]], gbuf[h], semg[h])

    def scatter_desc(h):
        return pltpu.make_async_copy(sbuf[h], acc.at[rb[h]], sems[h])

    def build_idx(h, m):
        for g in range(_CH // 16):
            sl = pl.ds(g * 16, 16)
            ib[h][sl] = mb[m][0, sl] + coff

    def build_rows(h, m):
        for g in range(_CH // 16):
            sl = pl.ds(g * 16, 16)
            rb[h][sl] = mb[m][1, sl]

    def scale(h, m):
        def body(g, carry):
            v16 = vb[m][pl.ds(g * 16, 16)]
            for e in range(16):
                bidx = jnp.full((16,), e, jnp.int32)
                v = jnp.take_along_axis(v16, bidx, axis=0,
                                        mode="promise_in_bounds")
                for j in range(_HD // 16):
                    sl = pl.ds(j * 16, 16)
                    sbuf[h][g * 16 + e, sl] = gbuf[h][g * 16 + e, sl] * v
            return carry

        lax.fori_loop(0, 8, body, 0)

    for j in range(6):
        meta_start(j, j)
    for u in range(4):
        meta_wait(u, u)
        build_idx(u, u)
        gather_desc(u).start()

    def zrow(r, carry):
        for j in range(_HD // 16):
            s0[r, pl.ds(j * 16, 16)] = jnp.zeros((16,), jnp.float32)
        return carry

    lax.fori_loop(0, _CH, zrow, 0)
    base = pl.multiple_of(s * _RPT, 8)
    for k in range(_RPT // _CH):
        pltpu.sync_copy(s0, acc.at[pl.ds(base + k * _CH, _CH)])
    plsc.subcore_barrier()

    def step(k, carry):
        for u in range(8):
            ch = 8 * k + u
            h = u % 4
            gather_desc(h).wait()

            @pl.when(ch >= 4)
            def _():
                scatter_desc(h).wait()

            @pl.when(ch + 6 < _NCH)
            def _():
                meta_start(ch + 6, (u + 6) % 8)

            scale(h, u)
            build_rows(h, u)

            @pl.when(ch + 4 < _NCH)
            def _():
                meta_wait(ch + 4, (u + 4) % 8)
                build_idx(h, (u + 4) % 8)
                gather_desc(h).start()

            scatter_desc(h).start(add=True)
        return carry

    lax.fori_loop(0, _NCH // 8, step, 0)
    for u in range(4):
        scatter_desc(u).wait()
    plsc.subcore_barrier()
    pltpu.sync_copy(acc.at[pl.ds(base, _RPT)],
                    out_hbm.at[c, pl.ds(base, _RPT)])


@functools.cache
def _make_spmv_call():
    return pl.kernel(
        _spmv_body,
        out_type=jax.ShapeDtypeStruct((_NC, _NP, _HD), jnp.float32),
        mesh=plsc.VectorSubcoreMesh(core_axis_name="c", subcore_axis_name="s",
                                    num_cores=_NC, num_subcores=_NS),
        scratch_types=(
            [pltpu.VMEM((2, _CH), jnp.int32)] * 8
            + [pltpu.VMEM((_CH,), jnp.float32)] * 8
            + [pltpu.VMEM((_CH, _HD), jnp.float32)] * 8
            + [pltpu.VMEM((_CH,), jnp.int32)] * 8
            + [pltpu.VMEM_SHARED((_NP, _HD), jnp.float32)]
            + [pltpu.SemaphoreType.DMA] * 16
        ),
        compiler_params=pltpu.CompilerParams(use_tc_tiling_on_sc=False),
        name="spmv_sc",
    )


def _spmv_call(*args):
    return _make_spmv_call()(*args)


def _elu(v):
    return jnp.where(v > 0, v, jnp.exp(jnp.minimum(v, 0.0)) - 1.0)


def _stats(y):
    m = jnp.sum(y, axis=0, keepdims=True) * (1.0 / _N)
    v = jnp.sum(y * y, axis=0, keepdims=True) * (1.0 / _N) - m * m
    return m, v


def _fold(m, v, g, b):
    s = g * lax.rsqrt(v + _EPS)
    return s, b - m * s


def _emit_y(y_ref, e):
    y_ref[0:_N, :] = e[:, :_HD]
    y_ref[_N:2 * _N, :] = e[:, _HD:]


def _read_y(y_ref):
    return jnp.concatenate([y_ref[0:_N], y_ref[_N:2 * _N]], axis=1)


def _read_other(part_ref):
    return jnp.concatenate([part_ref[0, :_N], part_ref[1, :_N]], axis=1)


def _seg0_body(inp_ref, w_ref, b_ref, x0_ref, y_ref):
    x = jnp.dot(inp_ref[...], w_ref[...],
                preferred_element_type=jnp.float32) + b_ref[...]
    x0_ref[...] = x
    _emit_y(y_ref, _elu(x))


def _sub_even(y, other, wt_ref, wb_ref, fcb_ref, g1_ref, b1_ref, g2_ref, b2_ref):
    m1, v1 = _stats(y)
    m2, v2 = _stats(other)
    s1, sh1 = _fold(m1, v1, g1_ref[...], b1_ref[...])
    s2, sh2 = _fold(m2, v2, g2_ref[...], b2_ref[...])
    sh = (jnp.dot(sh1, wt_ref[...], preferred_element_type=jnp.float32)
          + jnp.dot(sh2, wb_ref[...], preferred_element_type=jnp.float32)
          + fcb_ref[...])
    return (jnp.dot(y * s1, wt_ref[...], preferred_element_type=jnp.float32)
            + jnp.dot(other * s2, wb_ref[...], preferred_element_type=jnp.float32)
            + sh)


def _sub_odd(x, wt_ref, wb_ref, fcb_ref, g1_ref, b1_ref, b2_ref):
    y = _elu(x)
    m1, v1 = _stats(y)
    s1, sh1 = _fold(m1, v1, g1_ref[...], b1_ref[...])
    sh = (jnp.dot(sh1, wt_ref[...], preferred_element_type=jnp.float32)
          + jnp.dot(b2_ref[...], wb_ref[...], preferred_element_type=jnp.float32)
          + fcb_ref[...])
    return jnp.dot(y * s1, wt_ref[...], preferred_element_type=jnp.float32) + sh


def _seg_mid_body(y_ref, part_ref, wt_ref, wb_ref, fcb_ref,
                  g1_ref, b1_ref, g2_ref, b2_ref, y2_ref):
    x = _sub_even(_read_y(y_ref), _read_other(part_ref), wt_ref, wb_ref,
                  fcb_ref, g1_ref, b1_ref, g2_ref, b2_ref)
    _emit_y(y2_ref, _elu(x))


def _seg_fin_body(y2_ref, part_ref, res_ref, wt_ref, wb_ref, fcb_ref,
                  g1_ref, b1_ref, g2_ref, b2_ref,
                  owt0_ref, owb0_ref, ofcb0_ref, og0_ref, obt0_ref, obb0_ref,
                  owt1_ref, owb1_ref, ofcb1_ref, og1_ref, obt1_ref, obb1_ref,
                  xn_ref, yn_ref):
    x = _sub_even(_read_y(y2_ref), _read_other(part_ref), wt_ref, wb_ref,
                  fcb_ref, g1_ref, b1_ref, g2_ref, b2_ref)
    x = x + res_ref[...]
    r2 = x
    x = _sub_odd(x, owt0_ref, owb0_ref, ofcb0_ref, og0_ref, obt0_ref, obb0_ref)
    x = _sub_odd(x, owt1_ref, owb1_ref, ofcb1_ref, og1_ref, obt1_ref, obb1_ref)
    x = x + r2
    xn_ref[...] = x
    _emit_y(yn_ref, _elu(x))


def _seg_last_body(y2_ref, part_ref, res_ref, wt_ref, wb_ref, fcb_ref,
                   g1_ref, b1_ref, g2_ref, b2_ref,
                   c2g_ref, c2b_ref, c2w_ref, c2bias_ref, out_ref):
    x = _sub_even(_read_y(y2_ref), _read_other(part_ref), wt_ref, wb_ref,
                  fcb_ref, g1_ref, b1_ref, g2_ref, b2_ref)
    x = x + res_ref[...]
    m, v = _stats(x)
    s, sh = _fold(m, v, c2g_ref[...], c2b_ref[...])
    out = (jnp.dot(x * s, c2w_ref[...], preferred_element_type=jnp.float32)
           + jnp.dot(sh, c2w_ref[...], preferred_element_type=jnp.float32)
           + c2bias_ref[...])
    out_ref[...] = _elu(out)


_f32 = jnp.float32
_nd = jax.ShapeDtypeStruct((_N, _D), _f32)
_ys = jax.ShapeDtypeStruct((2 * _N, _HD), _f32)

_seg0_call = pl.pallas_call(_seg0_body, out_shape=[_nd, _ys], name="seg0")
_seg_mid_call = pl.pallas_call(_seg_mid_body, out_shape=[_ys], name="seg_mid")
_seg_fin_call = pl.pallas_call(_seg_fin_body, out_shape=[_nd, _ys], name="seg_fin")
_seg_last_call = pl.pallas_call(_seg_last_body, out_shape=[_nd], name="seg_last")


def kernel(L, L_values, mask, inputs, conv1_W, conv1_b, rn_bn_gamma, rn_bn_beta,
           rn_fc_W, rn_fc_b, conv2_bn_gamma, conv2_bn_beta, conv2_W, conv2_b):
    f32 = jnp.float32
    row = L[0].astype(jnp.int32)
    col = L[1].astype(jnp.int32)
    vals = L_values.astype(f32)
    pad = _EPAD - _E
    colp = jnp.concatenate([col, jnp.zeros((pad,), jnp.int32)]).reshape(_NS, _NCH, _CH)
    rowp = jnp.concatenate([row, jnp.zeros((pad,), jnp.int32)]).reshape(_NS, _NCH, _CH)
    valp = jnp.concatenate([vals, jnp.zeros((pad,), f32)]).reshape(_NS, _NCH, _CH)
    colrow = jnp.stack([colp, rowp], axis=2)

    inp = jnp.pad(inputs[0].astype(f32), ((0, 0), (0, 2)))
    w1 = jnp.pad(conv1_W.astype(f32), ((0, 2), (0, 0)))

    def even_args(i, j):
        return (rn_fc_W[i, j, :_D, :], rn_fc_W[i, j, _D:, :],
                rn_fc_b[i, j].reshape(1, _D),
                rn_bn_gamma[i, j, :_D].reshape(1, _D),
                rn_bn_beta[i, j, :_D].reshape(1, _D),
                rn_bn_gamma[i, j, _D:].reshape(1, _D),
                rn_bn_beta[i, j, _D:].reshape(1, _D))

    def odd_args(i, j):
        return (rn_fc_W[i, j, :_D, :], rn_fc_W[i, j, _D:, :],
                rn_fc_b[i, j].reshape(1, _D),
                rn_bn_gamma[i, j, :_D].reshape(1, _D),
                rn_bn_beta[i, j, :_D].reshape(1, _D),
                rn_bn_beta[i, j, _D:].reshape(1, _D))

    x0, y = _seg0_call(inp, w1, conv1_b.reshape(1, _D))
    res = x0
    out = None
    for i in range(0, _NB, 2):
        part = _spmv_call(y, colrow, valp)
        (y2,) = _seg_mid_call(y, part, *even_args(i, 0))
        part2 = _spmv_call(y2, colrow, valp)
        if i + 1 < _NB:
            res, y = _seg_fin_call(y2, part2, res, *even_args(i, 1),
                                   *odd_args(i + 1, 0), *odd_args(i + 1, 1))
        else:
            c2w = jnp.pad(conv2_W.astype(f32), ((0, 0), (0, _D - conv2_W.shape[1])))
            c2bias = jnp.pad(conv2_b.astype(f32), (0, _D - conv2_b.shape[0]))
            (out,) = _seg_last_call(y2, part2, res, *even_args(i, 1),
                                    conv2_bn_gamma.reshape(1, _D),
                                    conv2_bn_beta.reshape(1, _D),
                                    c2w, c2bias.reshape(1, _D))
    return out[:, :1].reshape(1, _N, 1)
